# baseline (speedup 1.0000x reference)
import functools

import jax
import jax.numpy as jnp
from jax import lax
from jax.experimental import pallas as pl
from jax.experimental.pallas import tpu as pltpu
from jax.experimental.pallas import tpu_sc as plsc

_D = 64
_SCALE = 8.0
_NW = 32
_C = 128
_LANES = 16


def _build(n_total):
    per_w = n_total // _NW
    n_chunks = per_w // _C
    mesh = plsc.VectorSubcoreMesh(core_axis_name="c", subcore_axis_name="s")

    @functools.partial(
        pl.kernel,
        mesh=mesh,
        out_type=jax.ShapeDtypeStruct((n_total, _D), jnp.float32),
        scratch_types=[
            pltpu.VMEM((per_w,), jnp.int32),
            pltpu.VMEM((per_w,), jnp.int32),
            pltpu.VMEM((_C, 2 * _D), jnp.float32),
            pltpu.VMEM((_C, 2 * _D), jnp.float32),
            pltpu.VMEM((_C, _D), jnp.float32),
            pltpu.VMEM((_C, _D), jnp.float32),
            pltpu.SemaphoreType.DMA,
            pltpu.SemaphoreType.DMA,
            pltpu.SemaphoreType.DMA,
            pltpu.SemaphoreType.DMA,
        ],
    )
    def k(x_hbm, tp_hbm, out_hbm, idx_v, half_v, rb0, rb1, ob0, ob1,
          g0, g1, o0, o1):
        rbs = (rb0, rb1)
        obs = (ob0, ob1)
        gsems = (g0, g1)
        osems = (o0, o1)
        wid = lax.axis_index("s") * 2 + lax.axis_index("c")
        base = wid * per_w
        pltpu.sync_copy(x_hbm.at[pl.ds(base, per_w)], idx_v)

        def halve(g, carry):
            sl = pl.ds(g * _LANES, _LANES)
            v = idx_v[sl]
            half_v[sl] = (v & 1) * _D
            idx_v[sl] = v >> 1
            return carry

        lax.fori_loop(0, per_w // _LANES, halve, 0, unroll=4)

        def gather(j, rb, gsem):
            pltpu.async_copy(tp_hbm.at[idx_v.at[pl.ds(j * _C, _C)]], rb, gsem)

        def wait_gather(rb, gsem):
            pltpu.make_async_copy(tp_hbm.at[pl.ds(0, _C)], rb, gsem).wait()

        def select(j, rb, ob):
            def grp(g, carry):
                hv = half_v[pl.ds(j * _C + g * _LANES, _LANES)]
                for l in range(_LANES):
                    r = g * _LANES + l
                    h = hv[l]
                    for c in range(_D // _LANES):
                        sl = pl.ds(h + c * _LANES, _LANES)
                        ob[r, pl.ds(c * _LANES, _LANES)] = rb[r, sl] * _SCALE
                return carry

            lax.fori_loop(0, _C // _LANES, grp, 0)

        def put(j, ob, osem):
            pltpu.async_copy(ob, out_hbm.at[pl.ds(base + j * _C, _C)], osem)

        def wait_put(ob, osem):
            pltpu.make_async_copy(ob, out_hbm.at[pl.ds(0, _C)], osem).wait()

        gather(0, rbs[0], gsems[0])

        def body(j, carry):
            def go(rb, ob, gsem, osem, nrb, ngsem):
                @pl.when(j + 1 < n_chunks)
                def _():
                    gather(j + 1, nrb, ngsem)

                wait_gather(rb, gsem)

                @pl.when(j >= 2)
                def _():
                    wait_put(ob, osem)

                select(j, rb, ob)
                put(j, ob, osem)

            lax.cond(
                lax.rem(j, 2) == 0,
                lambda: go(rbs[0], obs[0], gsems[0], osems[0], rbs[1], gsems[1]),
                lambda: go(rbs[1], obs[1], gsems[1], osems[1], rbs[0], gsems[0]),
            )
            return carry

        lax.fori_loop(0, n_chunks, body, 0)
        wait_put(obs[0], osems[0])
        wait_put(obs[1], osems[1])

    return k


def kernel(x, table):
    b, t = x.shape
    n_total = b * t
    xf = x.reshape(n_total)
    tp = table.reshape(table.shape[0] // 2, 2 * _D)
    out = _build(n_total)(xf, tp)
    return out.reshape(b, t, _D)


# speedup vs baseline: 1.6611x; 1.6611x over previous
import functools

import jax
import jax.numpy as jnp
from jax import lax
from jax.experimental import pallas as pl
from jax.experimental.pallas import tpu as pltpu
from jax.experimental.pallas import tpu_sc as plsc

_D = 64
_SCALE = 8.0
_NW = 32
_C = 256
_LANES = 16


def _build(n_total):
    per_w = n_total // _NW
    n_chunks = per_w // _C
    mesh = plsc.VectorSubcoreMesh(core_axis_name="c", subcore_axis_name="s")

    @functools.partial(
        pl.kernel,
        mesh=mesh,
        out_type=jax.ShapeDtypeStruct((n_total, _D), jnp.float32),
        scratch_types=[
            pltpu.VMEM((per_w,), jnp.int32),
            pltpu.VMEM((_C, _D), jnp.float32),
            pltpu.VMEM((_C, _D), jnp.float32),
            pltpu.SemaphoreType.DMA,
            pltpu.SemaphoreType.DMA,
            pltpu.SemaphoreType.DMA,
            pltpu.SemaphoreType.DMA,
        ],
    )
    def k(x_hbm, table_hbm, out_hbm, idx_v, rb0, rb1, g0, g1, o0, o1):
        rbs = (rb0, rb1)
        gsems = (g0, g1)
        osems = (o0, o1)
        wid = lax.axis_index("s") * 2 + lax.axis_index("c")
        base = wid * per_w
        pltpu.sync_copy(x_hbm.at[pl.ds(base, per_w)], idx_v)

        def enqueue(j, rb, gsem):
            def grp(g, carry):
                v = idx_v[pl.ds(j * _C + g * _LANES, _LANES)]
                for l in range(_LANES):
                    s = v[l]
                    pltpu.async_copy(
                        table_hbm.at[pl.ds(s, 1)],
                        rb.at[pl.ds(g * _LANES + l, 1)],
                        gsem,
                    )
                return carry

            lax.fori_loop(0, _C // _LANES, grp, 0)

        def drain(rb, gsem):
            pltpu.make_async_copy(table_hbm.at[pl.ds(0, _C)], rb, gsem).wait()

        def scale(rb):
            def row(r, carry):
                for c in range(_D // _LANES):
                    sl = pl.ds(c * _LANES, _LANES)
                    rb[r, sl] = rb[r, sl] * _SCALE
                return carry

            lax.fori_loop(0, _C, row, 0, unroll=4)

        def put(j, rb, osem):
            pltpu.async_copy(rb, out_hbm.at[pl.ds(base + j * _C, _C)], osem)

        def wait_put(rb, osem):
            pltpu.make_async_copy(rb, out_hbm.at[pl.ds(0, _C)], osem).wait()

        enqueue(0, rbs[0], gsems[0])

        def body(j, carry):
            def go(rb, gsem, osem, nrb, ngsem, nosem):
                @pl.when(j + 1 < n_chunks)
                def _():
                    @pl.when(j >= 1)
                    def _():
                        wait_put(nrb, nosem)

                    enqueue(j + 1, nrb, ngsem)

                drain(rb, gsem)
                scale(rb)
                put(j, rb, osem)

            lax.cond(
                lax.rem(j, 2) == 0,
                lambda: go(rbs[0], gsems[0], osems[0], rbs[1], gsems[1], osems[1]),
                lambda: go(rbs[1], gsems[1], osems[1], rbs[0], gsems[0], osems[0]),
            )
            return carry

        lax.fori_loop(0, n_chunks, body, 0)
        wait_put(rbs[0], osems[0])
        wait_put(rbs[1], osems[1])

    return k


def kernel(x, table):
    b, t = x.shape
    n_total = b * t
    xf = x.reshape(n_total)
    out = _build(n_total)(xf, table)
    return out.reshape(b, t, _D)
